# baseline (speedup 1.0000x reference)
import functools

import jax
import jax.numpy as jnp
from jax import lax
from jax.experimental import pallas as pl
from jax.experimental.pallas import tpu as pltpu
from jax.experimental.pallas import tpu_sc as plsc

B = 16384
D = 64
NC = 2
NS = 16
NW = NC * NS
BPW = B // NW
CH = 128
NCHK = BPW // CH

_mesh = plsc.VectorSubcoreMesh(core_axis_name="c", subcore_axis_name="s")


@functools.partial(
    pl.kernel,
    mesh=_mesh,
    out_type=jax.ShapeDtypeStruct((B,), jnp.float32),
    scratch_types=[
        pltpu.VMEM((NCHK, CH), jnp.int32),
        pltpu.VMEM((NCHK, CH), jnp.int32),
        pltpu.VMEM((2, CH, D), jnp.float32),
        pltpu.VMEM((2, CH, D), jnp.float32),
        pltpu.VMEM((BPW,), jnp.float32),
        pltpu.SemaphoreType.DMA,
        pltpu.SemaphoreType.DMA,
    ],
    compiler_params=pltpu.CompilerParams(
        needs_layout_passes=False, use_tc_tiling_on_sc=False),
)
def _sc_dot(uidx_hbm, iidx_hbm, utab_hbm, itab_hbm, out_hbm,
            uidx_v, iidx_v, urows_v, irows_v, out_v, sem0, sem1):
    wid = lax.axis_index("s") * NC + lax.axis_index("c")
    base = wid * BPW
    sems = (sem0, sem1)

    for j in range(NCHK):
        pltpu.sync_copy(uidx_hbm.at[pl.ds(base + j * CH, CH)], uidx_v.at[j])
        pltpu.sync_copy(iidx_hbm.at[pl.ds(base + j * CH, CH)], iidx_v.at[j])

    def fire(c, slot):
        pltpu.make_async_copy(
            utab_hbm.at[uidx_v.at[c]], urows_v.at[slot], sems[slot]).start()
        pltpu.make_async_copy(
            itab_hbm.at[iidx_v.at[c]], irows_v.at[slot], sems[slot]).start()

    def drain(c, slot):
        pltpu.make_async_copy(
            utab_hbm.at[uidx_v.at[c]], urows_v.at[slot], sems[slot]).wait()
        pltpu.make_async_copy(
            itab_hbm.at[iidx_v.at[c]], irows_v.at[slot], sems[slot]).wait()

    iota16 = lax.iota(jnp.int32, 16)

    fire(0, 0)
    for c in range(NCHK):
        slot = c % 2
        if c + 1 < NCHK:
            fire(c + 1, 1 - slot)
        drain(c, slot)

        def body(g, carry, slot=slot, cbase=c * CH):
            lb = g * 16
            s = jnp.zeros((16,), jnp.float32)
            for j in range(16):
                r = lb + j
                acc = (urows_v[slot, r, pl.ds(0, 16)]
                       * irows_v[slot, r, pl.ds(0, 16)])
                for q in range(1, D // 16):
                    acc = acc + (urows_v[slot, r, pl.ds(q * 16, 16)]
                                 * irows_v[slot, r, pl.ds(q * 16, 16)])
                tot = jnp.sum(acc)
                s = lax.select(iota16 == j, lax.broadcast(tot, (16,)), s)
            out_v[pl.ds(cbase + lb, 16)] = s
            return carry

        lax.fori_loop(0, CH // 16, body, 0)

    pltpu.sync_copy(out_v, out_hbm.at[pl.ds(base, BPW)])


def kernel(user_input, item_input, user_table, item_table):
    out = _sc_dot(user_input, item_input, user_table, item_table)
    return out.reshape(B, 1)


# speedup vs baseline: 1.5754x; 1.5754x over previous
import functools

import jax
import jax.numpy as jnp
from jax import lax
from jax.experimental import pallas as pl
from jax.experimental.pallas import tpu as pltpu
from jax.experimental.pallas import tpu_sc as plsc

B = 16384
D = 64
NC = 2
NS = 16
NW = NC * NS
BPW = B // NW
CH = 128
NCHK = BPW // CH

_mesh = plsc.VectorSubcoreMesh(core_axis_name="c", subcore_axis_name="s")


@functools.partial(
    pl.kernel,
    mesh=_mesh,
    out_type=jax.ShapeDtypeStruct((B,), jnp.float32),
    scratch_types=[
        pltpu.VMEM((BPW,), jnp.int32),
        pltpu.VMEM((BPW,), jnp.int32),
        pltpu.VMEM((2, CH, D), jnp.float32),
        pltpu.VMEM((2, CH, D), jnp.float32),
        pltpu.VMEM((BPW,), jnp.float32),
        pltpu.SemaphoreType.DMA,
        pltpu.SemaphoreType.DMA,
    ],
    compiler_params=pltpu.CompilerParams(
        needs_layout_passes=False, use_tc_tiling_on_sc=True),
)
def _sc_dot(uidx_hbm, iidx_hbm, utab_hbm, itab_hbm, out_hbm,
            uidx_v, iidx_v, urows_v, irows_v, out_v, sem0, sem1):
    wid = lax.axis_index("s") * NC + lax.axis_index("c")
    base = wid * BPW
    sems = (sem0, sem1)

    pltpu.sync_copy(uidx_hbm.at[pl.ds(base, BPW)], uidx_v)
    pltpu.sync_copy(iidx_hbm.at[pl.ds(base, BPW)], iidx_v)

    def fire(c, slot):
        cbase = c * CH

        def dma_body(g, carry, slot=slot, cbase=cbase):
            rb = cbase + g * 16
            lb = g * 16
            uvec = uidx_v[pl.ds(rb, 16)]
            ivec = iidx_v[pl.ds(rb, 16)]
            for j in range(16):
                iu = uvec[j]
                ii = ivec[j]
                pltpu.make_async_copy(
                    utab_hbm.at[pl.ds(iu, 1)],
                    urows_v.at[slot].at[pl.ds(lb + j, 1)], sems[slot]).start()
                pltpu.make_async_copy(
                    itab_hbm.at[pl.ds(ii, 1)],
                    irows_v.at[slot].at[pl.ds(lb + j, 1)], sems[slot]).start()
            return carry

        lax.fori_loop(0, CH // 16, dma_body, 0)

    def drain(slot):
        pltpu.make_async_copy(
            utab_hbm.at[pl.ds(0, CH)], urows_v.at[slot], sems[slot]).wait()
        pltpu.make_async_copy(
            itab_hbm.at[pl.ds(0, CH)], irows_v.at[slot], sems[slot]).wait()

    iota16 = lax.iota(jnp.int32, 16)

    fire(0, 0)
    for c in range(NCHK):
        slot = c % 2
        if c + 1 < NCHK:
            fire(c + 1, 1 - slot)
        drain(slot)

        def body(g, carry, slot=slot, cbase=c * CH):
            lb = g * 16
            s = jnp.zeros((16,), jnp.float32)
            for j in range(16):
                r = lb + j
                acc = (urows_v[slot, r, pl.ds(0, 16)]
                       * irows_v[slot, r, pl.ds(0, 16)])
                for q in range(1, D // 16):
                    acc = acc + (urows_v[slot, r, pl.ds(q * 16, 16)]
                                 * irows_v[slot, r, pl.ds(q * 16, 16)])
                tot = jnp.sum(acc)
                s = lax.select(iota16 == j, lax.broadcast(tot, (16,)), s)
            out_v[pl.ds(cbase + lb, 16)] = s
            return carry

        lax.fori_loop(0, CH // 16, body, 0)

    pltpu.sync_copy(out_v, out_hbm.at[pl.ds(base, BPW)])


def kernel(user_input, item_input, user_table, item_table):
    out = _sc_dot(user_input, item_input, user_table, item_table)
    return out.reshape(B, 1)
